# Initial kernel scaffold; baseline (speedup 1.0000x reference)
import jax
import jax.numpy as jnp
from jax.experimental import pallas as pl


def kernel(logits, uniform_noise):
    raise NotImplementedError("write your pallas kernel here")


# speedup vs baseline: 15.0789x; 15.0789x over previous
import functools

import jax
import jax.numpy as jnp
from jax import lax
from jax.experimental import pallas as pl
from jax.experimental.pallas import tpu as pltpu
from jax.experimental.pallas import tpu_sc as plsc

_B = 32
_V = 1_000_000
_TEMP = 0.8
_TOPK = 40
_TOPP = 0.9

_CHUNK = 20_000
_NCHUNK = _V // _CHUNK
_BLK = 400
_BVR = _BLK // 16
_NBLK = _CHUNK // _BLK
_CAP = 1552
_CVR = _CAP // 16
_TRIG = 1024
_NOUT = 128
_NSM = 144
_KEEP = 64
_PAD = -1e30


def _count_ge(ref, nvr, t):
    def body(k, acc):
        v = ref[pl.ds(k * 16, 16)]
        return acc + jnp.where(v >= t, 1, 0).astype(jnp.int32)
    acc = lax.fori_loop(0, nvr, body, jnp.zeros((16,), jnp.int32))
    return jnp.sum(acc)


def _search(ref, nvr, target):
    def it(_, lh):
        lo, hi = lh
        mid = (lo + hi) * jnp.float32(0.5)
        ok = _count_ge(ref, nvr, mid) >= target
        return jnp.where(ok, mid, lo), jnp.where(ok, hi, mid)
    lo, _ = lax.fori_loop(
        0, 22, it, (jnp.float32(-100.0), jnp.float32(100.0)))
    return lo


def _popcount(mask):
    return jnp.max(plsc.all_reduce_population_count(mask))


def _sc_body(logits_ref, noise2_ref, val_out, idx_out, u_out,
             buf0, buf1, cand_v, cand_i, small_v, small_i, small_hi,
             small_u, rows_u, sem0, sem1, gsem):
    row = lax.axis_index("s") * 2 + lax.axis_index("c")

    def dma(g, buf, sem):
        off = pl.multiple_of(g * _CHUNK, 8)
        return pltpu.make_async_copy(
            logits_ref.at[row, pl.ds(off, _CHUNK)], buf, sem)

    def init_cand(k, _):
        cand_v[pl.ds(k * 16, 16)] = jnp.full((16,), _PAD, jnp.float32)
        return 0
    lax.fori_loop(0, _CVR, init_cand, 0)

    def compact(tp):
        _, p = tp
        t2 = _search(cand_v, _CVR, _KEEP)
        def mv(k, wp):
            v = cand_v[pl.ds(k * 16, 16)]
            iv = cand_i[pl.ds(k * 16, 16)]
            msk = v >= t2
            plsc.store_compressed(cand_v.at[pl.ds(wp, 16)], v, mask=msk)
            plsc.store_compressed(cand_i.at[pl.ds(wp, 16)], iv, mask=msk)
            return wp + _popcount(msk)
        wp = lax.fori_loop(0, _CVR, mv, jnp.int32(0))
        nclr = (p - wp) // 16 + 2
        def clr(k, _):
            off = jnp.minimum(wp + k * 16, _CAP - 16)
            cand_v[pl.ds(off, 16)] = jnp.full((16,), _PAD, jnp.float32)
            return 0
        lax.fori_loop(0, nclr, clr, 0)
        return t2, wp

    lane = lax.iota(jnp.int32, 16)

    def process_chunk(buf, g, t, p):
        chunk_base = g * _CHUNK
        def blk(b, carry):
            t, p = carry
            base = b * _BLK
            vs = [buf[pl.ds(base + j * 16, 16)] for j in range(_BVR)]
            m = vs[0]
            for j in range(1, _BVR):
                m = jnp.maximum(m, vs[j])
            hit = jnp.any(m >= t)
            def do(tp):
                t, p = tp
                gbase = chunk_base + base
                p2 = p
                for j in range(_BVR):
                    msk = vs[j] >= t
                    plsc.store_compressed(
                        cand_v.at[pl.ds(p2, 16)], vs[j], mask=msk)
                    plsc.store_compressed(
                        cand_i.at[pl.ds(p2, 16)], gbase + j * 16 + lane,
                        mask=msk)
                    p2 = p2 + _popcount(msk)
                return lax.cond(p2 > _TRIG, compact, lambda x: x, (t, p2))
            return lax.cond(hit, do, lambda x: x, (t, p))
        return lax.fori_loop(0, _NBLK, blk, (t, p))

    dma(0, buf0, sem0).start()
    def two_chunks(i, carry):
        t, p = carry
        g0 = 2 * i
        dma(g0, buf0, sem0).wait()
        dma(g0 + 1, buf1, sem1).start()
        t, p = process_chunk(buf0, g0, t, p)
        dma(g0 + 1, buf1, sem1).wait()
        @pl.when(i + 1 < _NCHUNK // 2)
        def _():
            dma(g0 + 2, buf0, sem0).start()
        t, p = process_chunk(buf1, g0 + 1, t, p)
        return t, p
    t, p = lax.fori_loop(0, _NCHUNK // 2, two_chunks,
                         (jnp.float32(-3e38), jnp.int32(0)))

    tf = _search(cand_v, _CVR, _KEEP)
    for k in range(_NSM // 16):
        small_v[pl.ds(k * 16, 16)] = jnp.full((16,), _PAD, jnp.float32)
        small_i[pl.ds(k * 16, 16)] = jnp.zeros((16,), jnp.int32)
    def mv2(k, wp):
        v = cand_v[pl.ds(k * 16, 16)]
        iv = cand_i[pl.ds(k * 16, 16)]
        msk = v >= tf
        off = jnp.minimum(wp, _NOUT)
        plsc.store_compressed(small_v.at[pl.ds(off, 16)], v, mask=msk)
        plsc.store_compressed(small_i.at[pl.ds(off, 16)], iv, mask=msk)
        return wp + _popcount(msk)
    lax.fori_loop(0, _CVR, mv2, jnp.int32(0))

    rowbase = row * (_V // 16)
    for g in range(_NOUT // 16):
        iv = small_i[pl.ds(g * 16, 16)]
        small_hi[pl.ds(g * 16, 16)] = rowbase + jnp.right_shift(iv, 4)
    pltpu.make_async_copy(noise2_ref.at[small_hi], rows_u, gsem).start()
    pltpu.make_async_copy(noise2_ref.at[small_hi], rows_u, gsem).wait()
    for g in range(_NOUT // 16):
        ilo = jnp.bitwise_and(small_i[pl.ds(g * 16, 16)], 15)
        r = plsc.load_gather(rows_u, [g * 16 + lane, ilo])
        small_u[pl.ds(g * 16, 16)] = r

    pltpu.sync_copy(small_v.at[pl.ds(0, _NOUT)], val_out.at[row])
    pltpu.sync_copy(small_i.at[pl.ds(0, _NOUT)], idx_out.at[row])
    pltpu.sync_copy(small_u, u_out.at[row])


@functools.cache
def _sc_topk():
  return pl.kernel(
    _sc_body,
    out_type=[
        jax.ShapeDtypeStruct((_B, _NOUT), jnp.float32),
        jax.ShapeDtypeStruct((_B, _NOUT), jnp.int32),
        jax.ShapeDtypeStruct((_B, _NOUT), jnp.float32),
    ],
    mesh=plsc.VectorSubcoreMesh(core_axis_name="c", subcore_axis_name="s"),
    compiler_params=pltpu.CompilerParams(use_tc_tiling_on_sc=False,
                                         needs_layout_passes=False),
    scratch_types=[
        pltpu.VMEM((_CHUNK,), jnp.float32),
        pltpu.VMEM((_CHUNK,), jnp.float32),
        pltpu.VMEM((_CAP,), jnp.float32),
        pltpu.VMEM((_CAP,), jnp.int32),
        pltpu.VMEM((_NSM,), jnp.float32),
        pltpu.VMEM((_NSM,), jnp.int32),
        pltpu.VMEM((_NOUT,), jnp.int32),
        pltpu.VMEM((_NOUT,), jnp.float32),
        pltpu.VMEM((_NOUT, 16), jnp.float32),
        pltpu.SemaphoreType.DMA,
        pltpu.SemaphoreType.DMA,
        pltpu.SemaphoreType.DMA,
    ],
  )


def _tc_finalize_body(val_ref, idx_ref, u_ref, ids_ref, prob_ref):
    val = val_ref[...]
    idxs = idx_ref[...]
    u = u_ref[...]
    scaled = val / jnp.float32(_TEMP)
    lanes = lax.broadcasted_iota(jnp.int32, (_B, _NOUT), 1)
    big = jnp.int32(1 << 20)
    neg = jnp.float32(-1e9)

    cur = scaled
    svals = []
    for _ in range(_TOPK):
        m = jnp.max(cur, axis=1, keepdims=True)
        svals.append(m)
        first = jnp.min(jnp.where(cur == m, lanes, big), axis=1,
                        keepdims=True)
        cur = jnp.where(lanes == first, jnp.float32(-3e38), cur)
    s0, s39 = svals[0], svals[_TOPK - 1]

    e = jnp.exp(scaled - s0)
    denom1 = jnp.sum(jnp.where(scaled >= s39, e, 0.0), axis=1, keepdims=True)

    c = jnp.zeros_like(s0)
    kept = jnp.zeros_like(s0, dtype=jnp.int32)
    for j in range(_TOPK):
        pj = jnp.exp(svals[j] - s0) / denom1
        c = c + pj
        kept = kept + ((c - pj) < jnp.float32(_TOPP)).astype(jnp.int32)
    cut = svals[0]
    for j in range(1, _TOPK):
        cut = jnp.where(kept - 1 == j, svals[j], cut)

    filtered = jnp.where(scaled < cut, neg, scaled)
    denom2 = jnp.sum(jnp.where(scaled >= cut, e, 0.0), axis=1, keepdims=True)

    gumbel = -jnp.log(-jnp.log(u))
    comb = filtered + gumbel
    mx = jnp.max(comb, axis=1, keepdims=True)
    winner = jnp.min(jnp.where(comb == mx, lanes, big), axis=1, keepdims=True)
    win = lanes == winner
    wid = jnp.sum(jnp.where(win, idxs, 0), axis=1, keepdims=True)
    wval = jnp.sum(jnp.where(win, filtered, 0.0), axis=1, keepdims=True)
    prob = (wval - s0) - jnp.log(denom2)

    ids_ref[...] = jnp.broadcast_to(wid, (_B, _NOUT))
    prob_ref[...] = jnp.broadcast_to(prob, (_B, _NOUT))


def _tc_finalize(val, idx, u, interpret=False):
    return pl.pallas_call(
        _tc_finalize_body,
        out_shape=[
            jax.ShapeDtypeStruct((_B, _NOUT), jnp.int32),
            jax.ShapeDtypeStruct((_B, _NOUT), jnp.float32),
        ],
        interpret=interpret,
    )(val, idx, u)


def kernel(logits, uniform_noise):
    noise2 = uniform_noise.reshape(_B * (_V // 16), 16)
    val, idx, u = _sc_topk()(logits, noise2)
    ids2, prob2 = _tc_finalize(val, idx, u)
    return ids2[:, 0], prob2[:, 0]


# speedup vs baseline: 142.7885x; 9.4694x over previous
import functools

import jax
import jax.numpy as jnp
from jax import lax
from jax.experimental import pallas as pl
from jax.experimental.pallas import tpu as pltpu
from jax.experimental.pallas import tpu_sc as plsc

_B = 32
_V = 1_000_000
_TEMP = 0.8
_TOPK = 40
_TOPP = 0.9

_CW = 2048
_PART = 976 * 128
_NCH = _PART // _CW
_MAIN = 8 * _PART
_TAILW = _V - _MAIN
_BLKV = 32
_CAP = 1552
_CVR = _CAP // 16
_TRIG = 960
_KEEP = 48
_NOUT = 128
_NSM = 144
_PAD = -1e30


def _count_ge(ref, r8, t):
    def body(k, acc):
        v = ref[r8, pl.ds(k * 16, 16)]
        return acc + jnp.where(v >= t, 1, 0).astype(jnp.int32)
    acc = lax.fori_loop(0, _CVR, body, jnp.zeros((16,), jnp.int32))
    return jnp.sum(acc)


def _search(ref, r8, target):
    def it(_, lh):
        lo, hi = lh
        mid = (lo + hi) * jnp.float32(0.5)
        ok = _count_ge(ref, r8, mid) >= target
        return jnp.where(ok, mid, lo), jnp.where(ok, hi, mid)
    lo, _ = lax.fori_loop(
        0, 22, it, (jnp.float32(-100.0), jnp.float32(100.0)))
    return lo


def _popcount(mask):
    return jnp.max(plsc.all_reduce_population_count(mask))


def _sc_body(logits_ref, noise_ref, ltail_ref, ntail_ref,
             val_out, idx_out, u_out,
             lbuf0, lbuf1, nbuf0, nbuf1, cand_v, cand_i, cand_u,
             small_v, small_i, small_u, stage_v, stage_i, stage_u,
             tstate, pstate, sl0, sl1, sn0, sn1):
    wid = lax.axis_index("s") * 2 + lax.axis_index("c")
    tr = wid // 8
    pr = wid % 8
    rbase = pl.multiple_of(tr * 8, 8)
    c0 = pr * _PART
    lane = lax.iota(jnp.int32, 16)

    def dma_pair(coff, w, lb, nb, sl, sn):
        coff = pl.multiple_of(coff, 128)
        l = pltpu.make_async_copy(
            logits_ref.at[pl.ds(rbase, 8), pl.ds(coff, w)], lb, sl)
        n = pltpu.make_async_copy(
            noise_ref.at[pl.ds(rbase, 8), pl.ds(coff, w)], nb, sn)
        return l, n

    def start(g, lb, nb, sl, sn):
        l, n = dma_pair(c0 + g * _CW, _CW, lb, nb, sl, sn)
        l.start()
        n.start()

    def wait(g, lb, nb, sl, sn):
        l, n = dma_pair(c0 + g * _CW, _CW, lb, nb, sl, sn)
        l.wait()
        n.wait()

    def init_cand(k, _):
        cand_v[k // _CVR, pl.ds((k % _CVR) * 16, 16)] = jnp.full(
            (16,), _PAD, jnp.float32)
        return 0
    lax.fori_loop(0, 8 * _CVR, init_cand, 0)
    def init_state(r, _):
        tstate[r] = jnp.float32(-3e38)
        pstate[r] = jnp.int32(0)
        return 0
    lax.fori_loop(0, 8, init_state, 0)

    def compact(r8, tp):
        _, p = tp
        t2 = _search(cand_v, r8, _KEEP)
        def mv(k, wp):
            v = cand_v[r8, pl.ds(k * 16, 16)]
            iv = cand_i[r8, pl.ds(k * 16, 16)]
            uv = cand_u[r8, pl.ds(k * 16, 16)]
            msk = v >= t2
            plsc.store_compressed(cand_v.at[r8, pl.ds(wp, 16)], v, mask=msk)
            plsc.store_compressed(cand_i.at[r8, pl.ds(wp, 16)], iv, mask=msk)
            plsc.store_compressed(cand_u.at[r8, pl.ds(wp, 16)], uv, mask=msk)
            return wp + _popcount(msk)
        wp = lax.fori_loop(0, _CVR, mv, jnp.int32(0))
        nclr = (p - wp) // 16 + 2
        def clr(k, _):
            off = jnp.minimum(wp + k * 16, _CAP - 16)
            cand_v[r8, pl.ds(off, 16)] = jnp.full((16,), _PAD, jnp.float32)
            return 0
        lax.fori_loop(0, nclr, clr, 0)
        return t2, wp

    def append_block(r8, tp, vs, us, gidx0):
        t, p = tp
        p2 = p
        for j in range(len(vs)):
            msk = vs[j] >= t
            off = pl.ds(p2, 16)
            plsc.store_compressed(cand_v.at[r8, off], vs[j], mask=msk)
            plsc.store_compressed(
                cand_i.at[r8, off], gidx0 + j * 16 + lane, mask=msk)
            plsc.store_compressed(cand_u.at[r8, off], us[j], mask=msk)
            p2 = p2 + _popcount(msk)
        return lax.cond(p2 > _TRIG,
                        lambda tp2: compact(r8, tp2), lambda x: x, (t, p2))

    def scan_rows(lb, nb, nvr, colbase):
        nblk = (nvr + _BLKV - 1) // _BLKV
        def row_body(r8, _):
            t = tstate[r8]
            p = pstate[r8]
            def blk(b, tp):
                t, p = tp
                base = b * _BLKV * 16
                nv = min(_BLKV, nvr)
                vs = [lb[r8, pl.ds(base + j * 16, 16)] for j in range(nv)]
                m = vs[0]
                for j in range(1, nv):
                    m = jnp.maximum(m, vs[j])
                hit = jnp.any(m >= t)
                def do(tp):
                    us = [nb[r8, pl.ds(base + j * 16, 16)]
                          for j in range(nv)]
                    return append_block(r8, tp, vs, us, colbase + base)
                return lax.cond(hit, do, lambda x: x, (t, p))
            if nvr % _BLKV == 0:
                t, p = lax.fori_loop(0, nblk, blk, (t, p))
            else:
                t, p = lax.fori_loop(0, nblk - 1, blk, (t, p))
                base = (nblk - 1) * _BLKV * 16
                rem = nvr % _BLKV
                vs = [lb[r8, pl.ds(base + j * 16, 16)] for j in range(rem)]
                m = vs[0]
                for j in range(1, rem):
                    m = jnp.maximum(m, vs[j])
                hit = jnp.any(m >= t)
                def do2(tp):
                    us = [nb[r8, pl.ds(base + j * 16, 16)]
                          for j in range(rem)]
                    return append_block(r8, tp, vs, us, colbase + base)
                t, p = lax.cond(hit, do2, lambda x: x, (t, p))
            tstate[r8] = t
            pstate[r8] = p
            return 0
        lax.fori_loop(0, 8, row_body, 0)

    start(0, lbuf0, nbuf0, sl0, sn0)
    wait(0, lbuf0, nbuf0, sl0, sn0)
    start(1, lbuf1, nbuf1, sl1, sn1)
    scan_rows(lbuf0, nbuf0, _CW // 16, c0)

    def two_chunks(i, _):
        g = 1 + 2 * i
        wait(g, lbuf1, nbuf1, sl1, sn1)
        @pl.when(g + 1 < _NCH)
        def _():
            start(g + 1, lbuf0, nbuf0, sl0, sn0)
        scan_rows(lbuf1, nbuf1, _CW // 16, c0 + g * _CW)
        @pl.when(g + 1 < _NCH)
        def _():
            wait(g + 1, lbuf0, nbuf0, sl0, sn0)
            @pl.when(g + 2 < _NCH)
            def _():
                start(g + 2, lbuf1, nbuf1, sl1, sn1)
            scan_rows(lbuf0, nbuf0, _CW // 16, c0 + (g + 1) * _CW)
        return 0
    lax.fori_loop(0, _NCH // 2, two_chunks, 0)

    @pl.when(pr == 0)
    def _():
        coff = pl.multiple_of(_MAIN, 128)
        l1 = pltpu.make_async_copy(
            logits_ref.at[pl.ds(rbase, 8), pl.ds(coff, 512)],
            lbuf0.at[:, pl.ds(0, 512)], sl0)
        n1 = pltpu.make_async_copy(
            noise_ref.at[pl.ds(rbase, 8), pl.ds(coff, 512)],
            nbuf0.at[:, pl.ds(0, 512)], sn0)
        l2 = pltpu.make_async_copy(
            ltail_ref.at[pl.ds(rbase, 8), :],
            lbuf0.at[:, pl.ds(512, 128)], sl1)
        n2 = pltpu.make_async_copy(
            ntail_ref.at[pl.ds(rbase, 8), :],
            nbuf0.at[:, pl.ds(512, 128)], sn1)
        l1.start(); n1.start(); l2.start(); n2.start()
        l1.wait(); n1.wait(); l2.wait(); n2.wait()
        scan_rows(lbuf0, nbuf0, 640 // 16, _MAIN)

    def final_row(r8, _):
        tf = _search(cand_v, r8, _KEEP)
        for k in range(_NSM // 16):
            small_v[r8, pl.ds(k * 16, 16)] = jnp.full(
                (16,), _PAD, jnp.float32)
            small_i[r8, pl.ds(k * 16, 16)] = jnp.zeros((16,), jnp.int32)
            small_u[r8, pl.ds(k * 16, 16)] = jnp.full(
                (16,), 0.5, jnp.float32)
        def mv(k, wp):
            v = cand_v[r8, pl.ds(k * 16, 16)]
            iv = cand_i[r8, pl.ds(k * 16, 16)]
            uv = cand_u[r8, pl.ds(k * 16, 16)]
            msk = v >= tf
            off = jnp.minimum(wp, _NOUT)
            plsc.store_compressed(small_v.at[r8, pl.ds(off, 16)], v,
                                  mask=msk)
            plsc.store_compressed(small_i.at[r8, pl.ds(off, 16)], iv,
                                  mask=msk)
            plsc.store_compressed(small_u.at[r8, pl.ds(off, 16)], uv,
                                  mask=msk)
            return wp + _popcount(msk)
        lax.fori_loop(0, _CVR, mv, jnp.int32(0))
        for k in range(_NOUT // 16):
            stage_v[r8, pl.ds(k * 16, 16)] = small_v[r8, pl.ds(k * 16, 16)]
            stage_i[r8, pl.ds(k * 16, 16)] = small_i[r8, pl.ds(k * 16, 16)]
            stage_u[r8, pl.ds(k * 16, 16)] = small_u[r8, pl.ds(k * 16, 16)]
        return 0
    lax.fori_loop(0, 8, final_row, 0)

    cdst = pl.multiple_of(pr * _NOUT, 128)
    pltpu.sync_copy(stage_v, val_out.at[pl.ds(rbase, 8), pl.ds(cdst, _NOUT)])
    pltpu.sync_copy(stage_i, idx_out.at[pl.ds(rbase, 8), pl.ds(cdst, _NOUT)])
    pltpu.sync_copy(stage_u, u_out.at[pl.ds(rbase, 8), pl.ds(cdst, _NOUT)])


_MERGED = 8 * _NOUT


@functools.cache
def _sc_topk():
  return pl.kernel(
    _sc_body,
    out_type=[
        jax.ShapeDtypeStruct((_B, _MERGED), jnp.float32),
        jax.ShapeDtypeStruct((_B, _MERGED), jnp.int32),
        jax.ShapeDtypeStruct((_B, _MERGED), jnp.float32),
    ],
    mesh=plsc.VectorSubcoreMesh(core_axis_name="c", subcore_axis_name="s"),
    compiler_params=pltpu.CompilerParams(use_tc_tiling_on_sc=True,
                                         needs_layout_passes=False),
    scratch_types=[
        pltpu.VMEM((8, _CW), jnp.float32),
        pltpu.VMEM((8, _CW), jnp.float32),
        pltpu.VMEM((8, _CW), jnp.float32),
        pltpu.VMEM((8, _CW), jnp.float32),
        pltpu.VMEM((8, _CAP), jnp.float32),
        pltpu.VMEM((8, _CAP), jnp.int32),
        pltpu.VMEM((8, _CAP), jnp.float32),
        pltpu.VMEM((8, _NSM), jnp.float32),
        pltpu.VMEM((8, _NSM), jnp.int32),
        pltpu.VMEM((8, _NSM), jnp.float32),
        pltpu.VMEM((8, _NOUT), jnp.float32),
        pltpu.VMEM((8, _NOUT), jnp.int32),
        pltpu.VMEM((8, _NOUT), jnp.float32),
        pltpu.SMEM((8,), jnp.float32),
        pltpu.SMEM((8,), jnp.int32),
        pltpu.SemaphoreType.DMA,
        pltpu.SemaphoreType.DMA,
        pltpu.SemaphoreType.DMA,
        pltpu.SemaphoreType.DMA,
    ],
  )


def _tc_finalize_body(val_ref, idx_ref, u_ref, ids_ref, prob_ref):
    val = val_ref[...]
    idxs = idx_ref[...]
    u = u_ref[...]
    scaled = val / jnp.float32(_TEMP)
    lanes = lax.broadcasted_iota(jnp.int32, (_B, _MERGED), 1)
    big = jnp.int32(1 << 30)
    neg = jnp.float32(-1e9)

    cur = scaled
    svals = []
    for _ in range(_TOPK):
        m = jnp.max(cur, axis=1, keepdims=True)
        svals.append(m)
        first = jnp.min(jnp.where(cur == m, lanes, big), axis=1,
                        keepdims=True)
        cur = jnp.where(lanes == first, jnp.float32(-3e38), cur)
    s0, s39 = svals[0], svals[_TOPK - 1]

    e = jnp.exp(scaled - s0)
    denom1 = jnp.sum(jnp.where(scaled >= s39, e, 0.0), axis=1, keepdims=True)

    c = jnp.zeros_like(s0)
    kept = jnp.zeros_like(s0, dtype=jnp.int32)
    for j in range(_TOPK):
        pj = jnp.exp(svals[j] - s0) / denom1
        c = c + pj
        kept = kept + ((c - pj) < jnp.float32(_TOPP)).astype(jnp.int32)
    cut = svals[0]
    for j in range(1, _TOPK):
        cut = jnp.where(kept - 1 == j, svals[j], cut)

    filtered = jnp.where(scaled < cut, neg, scaled)
    denom2 = jnp.sum(jnp.where(scaled >= cut, e, 0.0), axis=1, keepdims=True)

    gumbel = -jnp.log(-jnp.log(u))
    comb = filtered + gumbel
    mx = jnp.max(comb, axis=1, keepdims=True)
    wid = jnp.min(jnp.where(comb == mx, idxs, big), axis=1, keepdims=True)
    win = (comb == mx) & (idxs == wid)
    wval = jnp.max(jnp.where(win, filtered, jnp.float32(-3e38)), axis=1,
                   keepdims=True)
    prob = (wval - s0) - jnp.log(denom2)

    ids_ref[...] = jnp.broadcast_to(wid, (_B, _MERGED))
    prob_ref[...] = jnp.broadcast_to(prob, (_B, _MERGED))


def _tc_finalize(val, idx, u, interpret=False):
    return pl.pallas_call(
        _tc_finalize_body,
        out_shape=[
            jax.ShapeDtypeStruct((_B, _MERGED), jnp.int32),
            jax.ShapeDtypeStruct((_B, _MERGED), jnp.float32),
        ],
        interpret=interpret,
    )(val, idx, u)


def kernel(logits, uniform_noise):
    ltail = jnp.pad(logits[:, _MAIN + 512:], ((0, 0), (0, 64)),
                    constant_values=_PAD)
    ntail = jnp.pad(uniform_noise[:, _MAIN + 512:], ((0, 0), (0, 64)),
                    constant_values=0.5)
    val, idx, u = _sc_topk()(logits, uniform_noise, ltail, ntail)
    ids2, prob2 = _tc_finalize(val, idx, u)
    return ids2[:, 0], prob2[:, 0]


# speedup vs baseline: 146.2830x; 1.0245x over previous
import functools

import jax
import jax.numpy as jnp
from jax import lax
from jax.experimental import pallas as pl
from jax.experimental.pallas import tpu as pltpu
from jax.experimental.pallas import tpu_sc as plsc

_B = 32
_V = 1_000_000
_TEMP = 0.8
_TOPK = 40
_TOPP = 0.9

_CW = 2048
_PART = 976 * 128
_NCH = _PART // _CW
_MAIN = 8 * _PART
_TAILW = _V - _MAIN
_BLKV = 32
_CAP = 1552
_CVR = _CAP // 16
_TRIG = 960
_KEEP = 48
_NOUT = 128
_NSM = 144
_PAD = -1e30


def _count_ge(ref, r8, t):
    def body(k, acc):
        v = ref[r8, pl.ds(k * 16, 16)]
        return acc + jnp.where(v >= t, 1, 0).astype(jnp.int32)
    acc = lax.fori_loop(0, _CVR, body, jnp.zeros((16,), jnp.int32))
    return jnp.sum(acc)


def _search(ref, r8, target):
    def it(_, lh):
        lo, hi = lh
        mid = (lo + hi) * jnp.float32(0.5)
        ok = _count_ge(ref, r8, mid) >= target
        return jnp.where(ok, mid, lo), jnp.where(ok, hi, mid)
    lo, _ = lax.fori_loop(
        0, 22, it, (jnp.float32(-100.0), jnp.float32(100.0)))
    return lo


def _popcount(mask):
    return jnp.max(plsc.all_reduce_population_count(mask))


def _sc_body(logits_ref, noise_ref, ltail_ref, ntail_ref,
             val_out, idx_out, u_out,
             lbuf0, lbuf1, nbuf0, nbuf1, cand_v, cand_i, cand_u,
             small_v, small_i, small_u, stage_v, stage_i, stage_u,
             tstate, pstate, sl0, sl1, sn0, sn1):
    wid = lax.axis_index("s") * 2 + lax.axis_index("c")
    tr = wid // 8
    pr = wid % 8
    rbase = pl.multiple_of(tr * 8, 8)
    c0 = pr * _PART
    lane = lax.iota(jnp.int32, 16)

    def dma_pair(coff, w, lb, nb, sl, sn):
        coff = pl.multiple_of(coff, 128)
        l = pltpu.make_async_copy(
            logits_ref.at[pl.ds(rbase, 8), pl.ds(coff, w)], lb, sl)
        n = pltpu.make_async_copy(
            noise_ref.at[pl.ds(rbase, 8), pl.ds(coff, w)], nb, sn)
        return l, n

    def start(g, lb, nb, sl, sn):
        l, n = dma_pair(c0 + g * _CW, _CW, lb, nb, sl, sn)
        l.start()
        n.start()

    def wait(g, lb, nb, sl, sn):
        l, n = dma_pair(c0 + g * _CW, _CW, lb, nb, sl, sn)
        l.wait()
        n.wait()

    def init_cand(k, _):
        cand_v[k // _CVR, pl.ds((k % _CVR) * 16, 16)] = jnp.full(
            (16,), _PAD, jnp.float32)
        return 0
    lax.fori_loop(0, 8 * _CVR, init_cand, 0)
    def init_state(r, _):
        tstate[r] = jnp.float32(-3e38)
        pstate[r] = jnp.int32(0)
        return 0
    lax.fori_loop(0, 8, init_state, 0)

    def compact(r8, tp):
        _, p = tp
        t2 = _search(cand_v, r8, _KEEP)
        def mv(k, wp):
            v = cand_v[r8, pl.ds(k * 16, 16)]
            iv = cand_i[r8, pl.ds(k * 16, 16)]
            uv = cand_u[r8, pl.ds(k * 16, 16)]
            msk = v >= t2
            plsc.store_compressed(cand_v.at[r8, pl.ds(wp, 16)], v, mask=msk)
            plsc.store_compressed(cand_i.at[r8, pl.ds(wp, 16)], iv, mask=msk)
            plsc.store_compressed(cand_u.at[r8, pl.ds(wp, 16)], uv, mask=msk)
            return wp + _popcount(msk)
        wp = lax.fori_loop(0, _CVR, mv, jnp.int32(0))
        nclr = (p - wp) // 16 + 2
        def clr(k, _):
            off = jnp.minimum(wp + k * 16, _CAP - 16)
            cand_v[r8, pl.ds(off, 16)] = jnp.full((16,), _PAD, jnp.float32)
            return 0
        lax.fori_loop(0, nclr, clr, 0)
        return t2, wp

    def append_block(r8, tp, vs, us, gidx0):
        t, p = tp
        p2 = p
        for j in range(len(vs)):
            msk = vs[j] >= t
            off = pl.ds(p2, 16)
            plsc.store_compressed(cand_v.at[r8, off], vs[j], mask=msk)
            plsc.store_compressed(
                cand_i.at[r8, off], gidx0 + j * 16 + lane, mask=msk)
            plsc.store_compressed(cand_u.at[r8, off], us[j], mask=msk)
            p2 = p2 + _popcount(msk)
        return lax.cond(p2 > _TRIG,
                        lambda tp2: compact(r8, tp2), lambda x: x, (t, p2))

    def scan_rows(lb, nb, nvr, colbase):
        nblk = (nvr + _BLKV - 1) // _BLKV
        def row_body(r8, _):
            t = tstate[r8]
            p = pstate[r8]
            def blk(b, tp):
                t, p = tp
                base = b * _BLKV * 16
                nv = min(_BLKV, nvr)
                m = lb[r8, pl.ds(base, 16)]
                for j in range(1, nv):
                    m = jnp.maximum(m, lb[r8, pl.ds(base + j * 16, 16)])
                hit = jnp.max(m) >= t
                def do(tp):
                    vs = [lb[r8, pl.ds(base + j * 16, 16)]
                          for j in range(nv)]
                    us = [nb[r8, pl.ds(base + j * 16, 16)]
                          for j in range(nv)]
                    return append_block(r8, tp, vs, us, colbase + base)
                return lax.cond(hit, do, lambda x: x, (t, p))
            if nvr % _BLKV == 0:
                t, p = lax.fori_loop(0, nblk, blk, (t, p))
            else:
                t, p = lax.fori_loop(0, nblk - 1, blk, (t, p))
                base = (nblk - 1) * _BLKV * 16
                rem = nvr % _BLKV
                m = lb[r8, pl.ds(base, 16)]
                for j in range(1, rem):
                    m = jnp.maximum(m, lb[r8, pl.ds(base + j * 16, 16)])
                hit = jnp.max(m) >= t
                def do2(tp):
                    vs = [lb[r8, pl.ds(base + j * 16, 16)]
                          for j in range(rem)]
                    us = [nb[r8, pl.ds(base + j * 16, 16)]
                          for j in range(rem)]
                    return append_block(r8, tp, vs, us, colbase + base)
                t, p = lax.cond(hit, do2, lambda x: x, (t, p))
            tstate[r8] = t
            pstate[r8] = p
            return 0
        lax.fori_loop(0, 8, row_body, 0)

    start(0, lbuf0, nbuf0, sl0, sn0)
    wait(0, lbuf0, nbuf0, sl0, sn0)
    start(1, lbuf1, nbuf1, sl1, sn1)
    scan_rows(lbuf0, nbuf0, _CW // 16, c0)

    def two_chunks(i, _):
        g = 1 + 2 * i
        wait(g, lbuf1, nbuf1, sl1, sn1)
        @pl.when(g + 1 < _NCH)
        def _():
            start(g + 1, lbuf0, nbuf0, sl0, sn0)
        scan_rows(lbuf1, nbuf1, _CW // 16, c0 + g * _CW)
        @pl.when(g + 1 < _NCH)
        def _():
            wait(g + 1, lbuf0, nbuf0, sl0, sn0)
            @pl.when(g + 2 < _NCH)
            def _():
                start(g + 2, lbuf1, nbuf1, sl1, sn1)
            scan_rows(lbuf0, nbuf0, _CW // 16, c0 + (g + 1) * _CW)
        return 0
    lax.fori_loop(0, _NCH // 2, two_chunks, 0)

    @pl.when(pr == 0)
    def _():
        coff = pl.multiple_of(_MAIN, 128)
        l1 = pltpu.make_async_copy(
            logits_ref.at[pl.ds(rbase, 8), pl.ds(coff, 512)],
            lbuf0.at[:, pl.ds(0, 512)], sl0)
        n1 = pltpu.make_async_copy(
            noise_ref.at[pl.ds(rbase, 8), pl.ds(coff, 512)],
            nbuf0.at[:, pl.ds(0, 512)], sn0)
        l2 = pltpu.make_async_copy(
            ltail_ref.at[pl.ds(rbase, 8), :],
            lbuf0.at[:, pl.ds(512, 128)], sl1)
        n2 = pltpu.make_async_copy(
            ntail_ref.at[pl.ds(rbase, 8), :],
            nbuf0.at[:, pl.ds(512, 128)], sn1)
        l1.start(); n1.start(); l2.start(); n2.start()
        l1.wait(); n1.wait(); l2.wait(); n2.wait()
        scan_rows(lbuf0, nbuf0, 640 // 16, _MAIN)

    def final_row(r8, _):
        tf = _search(cand_v, r8, _KEEP)
        for k in range(_NSM // 16):
            small_v[r8, pl.ds(k * 16, 16)] = jnp.full(
                (16,), _PAD, jnp.float32)
            small_i[r8, pl.ds(k * 16, 16)] = jnp.zeros((16,), jnp.int32)
            small_u[r8, pl.ds(k * 16, 16)] = jnp.full(
                (16,), 0.5, jnp.float32)
        def mv(k, wp):
            v = cand_v[r8, pl.ds(k * 16, 16)]
            iv = cand_i[r8, pl.ds(k * 16, 16)]
            uv = cand_u[r8, pl.ds(k * 16, 16)]
            msk = v >= tf
            off = jnp.minimum(wp, _NOUT)
            plsc.store_compressed(small_v.at[r8, pl.ds(off, 16)], v,
                                  mask=msk)
            plsc.store_compressed(small_i.at[r8, pl.ds(off, 16)], iv,
                                  mask=msk)
            plsc.store_compressed(small_u.at[r8, pl.ds(off, 16)], uv,
                                  mask=msk)
            return wp + _popcount(msk)
        lax.fori_loop(0, _CVR, mv, jnp.int32(0))
        for k in range(_NOUT // 16):
            stage_v[r8, pl.ds(k * 16, 16)] = small_v[r8, pl.ds(k * 16, 16)]
            stage_i[r8, pl.ds(k * 16, 16)] = small_i[r8, pl.ds(k * 16, 16)]
            stage_u[r8, pl.ds(k * 16, 16)] = small_u[r8, pl.ds(k * 16, 16)]
        return 0
    lax.fori_loop(0, 8, final_row, 0)

    cdst = pl.multiple_of(pr * _NOUT, 128)
    pltpu.sync_copy(stage_v, val_out.at[pl.ds(rbase, 8), pl.ds(cdst, _NOUT)])
    pltpu.sync_copy(stage_i, idx_out.at[pl.ds(rbase, 8), pl.ds(cdst, _NOUT)])
    pltpu.sync_copy(stage_u, u_out.at[pl.ds(rbase, 8), pl.ds(cdst, _NOUT)])


_MERGED = 8 * _NOUT


@functools.cache
def _sc_topk():
  return pl.kernel(
    _sc_body,
    out_type=[
        jax.ShapeDtypeStruct((_B, _MERGED), jnp.float32),
        jax.ShapeDtypeStruct((_B, _MERGED), jnp.int32),
        jax.ShapeDtypeStruct((_B, _MERGED), jnp.float32),
    ],
    mesh=plsc.VectorSubcoreMesh(core_axis_name="c", subcore_axis_name="s"),
    compiler_params=pltpu.CompilerParams(use_tc_tiling_on_sc=True,
                                         needs_layout_passes=False),
    scratch_types=[
        pltpu.VMEM((8, _CW), jnp.float32),
        pltpu.VMEM((8, _CW), jnp.float32),
        pltpu.VMEM((8, _CW), jnp.float32),
        pltpu.VMEM((8, _CW), jnp.float32),
        pltpu.VMEM((8, _CAP), jnp.float32),
        pltpu.VMEM((8, _CAP), jnp.int32),
        pltpu.VMEM((8, _CAP), jnp.float32),
        pltpu.VMEM((8, _NSM), jnp.float32),
        pltpu.VMEM((8, _NSM), jnp.int32),
        pltpu.VMEM((8, _NSM), jnp.float32),
        pltpu.VMEM((8, _NOUT), jnp.float32),
        pltpu.VMEM((8, _NOUT), jnp.int32),
        pltpu.VMEM((8, _NOUT), jnp.float32),
        pltpu.SMEM((8,), jnp.float32),
        pltpu.SMEM((8,), jnp.int32),
        pltpu.SemaphoreType.DMA,
        pltpu.SemaphoreType.DMA,
        pltpu.SemaphoreType.DMA,
        pltpu.SemaphoreType.DMA,
    ],
  )


def _tc_finalize_body(val_ref, idx_ref, u_ref, ids_ref, prob_ref):
    val = val_ref[...]
    idxs = idx_ref[...]
    u = u_ref[...]
    scaled = val / jnp.float32(_TEMP)
    lanes = lax.broadcasted_iota(jnp.int32, (_B, _MERGED), 1)
    big = jnp.int32(1 << 30)
    neg = jnp.float32(-1e9)

    cur = scaled
    svals = []
    for _ in range(_TOPK):
        m = jnp.max(cur, axis=1, keepdims=True)
        svals.append(m)
        first = jnp.min(jnp.where(cur == m, lanes, big), axis=1,
                        keepdims=True)
        cur = jnp.where(lanes == first, jnp.float32(-3e38), cur)
    s0, s39 = svals[0], svals[_TOPK - 1]

    e = jnp.exp(scaled - s0)
    denom1 = jnp.sum(jnp.where(scaled >= s39, e, 0.0), axis=1, keepdims=True)

    c = jnp.zeros_like(s0)
    kept = jnp.zeros_like(s0, dtype=jnp.int32)
    for j in range(_TOPK):
        pj = jnp.exp(svals[j] - s0) / denom1
        c = c + pj
        kept = kept + ((c - pj) < jnp.float32(_TOPP)).astype(jnp.int32)
    cut = svals[0]
    for j in range(1, _TOPK):
        cut = jnp.where(kept - 1 == j, svals[j], cut)

    filtered = jnp.where(scaled < cut, neg, scaled)
    denom2 = jnp.sum(jnp.where(scaled >= cut, e, 0.0), axis=1, keepdims=True)

    gumbel = -jnp.log(-jnp.log(u))
    comb = filtered + gumbel
    mx = jnp.max(comb, axis=1, keepdims=True)
    wid = jnp.min(jnp.where(comb == mx, idxs, big), axis=1, keepdims=True)
    win = (comb == mx) & (idxs == wid)
    wval = jnp.max(jnp.where(win, filtered, jnp.float32(-3e38)), axis=1,
                   keepdims=True)
    prob = (wval - s0) - jnp.log(denom2)

    ids_ref[...] = jnp.broadcast_to(wid, (_B, _MERGED))
    prob_ref[...] = jnp.broadcast_to(prob, (_B, _MERGED))


def _tc_finalize(val, idx, u, interpret=False):
    return pl.pallas_call(
        _tc_finalize_body,
        out_shape=[
            jax.ShapeDtypeStruct((_B, _MERGED), jnp.int32),
            jax.ShapeDtypeStruct((_B, _MERGED), jnp.float32),
        ],
        interpret=interpret,
    )(val, idx, u)


def kernel(logits, uniform_noise):
    ltail = jnp.pad(logits[:, _MAIN + 512:], ((0, 0), (0, 64)),
                    constant_values=_PAD)
    ntail = jnp.pad(uniform_noise[:, _MAIN + 512:], ((0, 0), (0, 64)),
                    constant_values=0.5)
    val, idx, u = _sc_topk()(logits, uniform_noise, ltail, ntail)
    ids2, prob2 = _tc_finalize(val, idx, u)
    return ids2[:, 0], prob2[:, 0]
